# baseline (speedup 1.0000x reference)
import functools

import jax
import jax.numpy as jnp
from jax import lax
from jax.experimental import pallas as pl
from jax.experimental.pallas import tpu as pltpu
from jax.experimental.pallas import tpu_sc as plsc

B, L, V, E, H = 1024, 50, 1000000, 64, 64
G4 = 4 * H

NC, NS = 2, 16
NW = NC * NS
B_PER_W = B // NW


def _sc_gather(emb, x):
    mesh = plsc.VectorSubcoreMesh(core_axis_name="c", subcore_axis_name="s")

    @functools.partial(
        pl.kernel,
        mesh=mesh,
        out_type=jax.ShapeDtypeStruct((B * L, E), jnp.float32),
        scratch_types=[
            pltpu.VMEM((B_PER_W, L), jnp.int32),
            pltpu.SMEM((8, L), jnp.int32),
            pltpu.SemaphoreType.DMA,
        ],
    )
    def gather_kernel(table_hbm, x_hbm, out_hbm, idx_v, idx_s, sem):
        wid = lax.axis_index("s") * NC + lax.axis_index("c")
        b0 = wid * B_PER_W
        pltpu.sync_copy(x_hbm.at[pl.ds(b0, B_PER_W)], idx_v)

        @pl.loop(0, B_PER_W // 8)
        def _(k8):
            pltpu.sync_copy(idx_v.at[pl.ds(k8 * 8, 8)], idx_s)

            @pl.loop(0, 8)
            def _(k):
                @pl.loop(0, L)
                def _(t):
                    row = idx_s[k, t]
                    pltpu.make_async_copy(
                        table_hbm.at[row],
                        out_hbm.at[t * B + b0 + k8 * 8 + k],
                        sem,
                    ).start()

        pltpu.make_async_copy(
            table_hbm.at[pl.ds(0, B_PER_W * L)],
            out_hbm.at[pl.ds(b0 * L, B_PER_W * L)],
            sem,
        ).wait()

    return gather_kernel(emb, x)


BC = 256


def _lstm_body(e_ref, wih_ref, whh_ref, b_ref, out_ref, xp_ref, acc_ref):
    e2 = e_ref[...].reshape(L * BC, E)
    xp_ref[...] = (
        jnp.dot(e2, wih_ref[...], preferred_element_type=jnp.float32) + b_ref[...]
    )

    def step(t, carry):
        h, c = carry
        gates = xp_ref[pl.ds(t * BC, BC), :] + jnp.dot(
            h, whh_ref[...], preferred_element_type=jnp.float32
        )
        i = jax.nn.sigmoid(gates[:, 0:H])
        f = jax.nn.sigmoid(gates[:, H : 2 * H])
        g = jnp.tanh(gates[:, 2 * H : 3 * H])
        o = jax.nn.sigmoid(gates[:, 3 * H :])
        c = f * c + i * g
        h = o * jnp.tanh(c)
        acc_ref[t] = h
        return (h, c)

    h0 = jnp.zeros((BC, H), jnp.float32)
    c0 = jnp.zeros((BC, H), jnp.float32)
    lax.fori_loop(0, L, step, (h0, c0))
    out_ref[...] = jnp.swapaxes(acc_ref[...], 0, 1)


def _lstm_tc(e_lbe, wih_t, whh_t, bias):
    return pl.pallas_call(
        _lstm_body,
        grid=(B // BC,),
        in_specs=[
            pl.BlockSpec((L, BC, E), lambda i: (0, i, 0)),
            pl.BlockSpec((E, G4), lambda i: (0, 0)),
            pl.BlockSpec((H, G4), lambda i: (0, 0)),
            pl.BlockSpec((1, G4), lambda i: (0, 0)),
        ],
        out_specs=pl.BlockSpec((BC, L, H), lambda i: (i, 0, 0)),
        out_shape=jax.ShapeDtypeStruct((B, L, H), jnp.float32),
        scratch_shapes=[
            pltpu.VMEM((L * BC, G4), jnp.float32),
            pltpu.VMEM((L, BC, H), jnp.float32),
        ],
    )(e_lbe, wih_t, whh_t, bias)


def kernel(x, emb, W_ih, W_hh, b_ih, b_hh):
    e = jnp.take(emb, x.T.reshape(-1), axis=0)
    bias = (b_ih + b_hh).reshape(1, G4)
    return _lstm_tc(e.reshape(L, B, E), W_ih.T, W_hh.T, bias)


# speedup vs baseline: 1.1130x; 1.1130x over previous
import functools

import jax
import jax.numpy as jnp
from jax import lax
from jax.experimental import pallas as pl
from jax.experimental.pallas import tpu as pltpu
from jax.experimental.pallas import tpu_sc as plsc

B, L, V, E, H = 1024, 50, 1000000, 64, 64
G4 = 4 * H


def _lstm_body(e_hbm, wih_ref, whh_ref, b_ref, out_hbm,
               e_buf, h_buf, in_sem, out_sem):
    def in_copy(t, slot):
        return pltpu.make_async_copy(
            e_hbm.at[pl.ds(t * B, B)], e_buf.at[slot], in_sem.at[slot]
        )

    def out_copy(t, slot):
        return pltpu.make_async_copy(
            h_buf.at[slot], out_hbm.at[:, t], out_sem.at[slot]
        )

    in_copy(0, 0).start()
    in_copy(1, 1).start()

    def gate(g, et, h):
        w_i = wih_ref[pl.ds(g * E, E), :]
        w_h = whh_ref[pl.ds(g * H, H), :]
        acc = jnp.dot(et, w_i, preferred_element_type=jnp.float32)
        acc += jnp.dot(h, w_h, preferred_element_type=jnp.float32)
        return acc + b_ref[g, :]

    def step(t, carry):
        h, c = carry
        eslot = lax.rem(t, 3)
        oslot = lax.rem(t, 2)
        in_copy(t, eslot).wait()
        et = e_buf[eslot]

        @pl.when(t + 2 < L)
        def _():
            in_copy(t + 2, lax.rem(t + 2, 3)).start()

        i = jax.nn.sigmoid(gate(0, et, h))
        f = jax.nn.sigmoid(gate(1, et, h))
        g = jnp.tanh(gate(2, et, h))
        o = jax.nn.sigmoid(gate(3, et, h))
        c = f * c + i * g
        h = o * jnp.tanh(c)

        @pl.when(t >= 2)
        def _():
            out_copy(t - 2, oslot).wait()

        h_buf[oslot] = h
        out_copy(t, oslot).start()
        return (h, c)

    h0 = jnp.zeros((B, H), jnp.float32)
    c0 = jnp.zeros((B, H), jnp.float32)
    lax.fori_loop(0, L, step, (h0, c0))
    out_copy(L - 2, 0).wait()
    out_copy(L - 1, 1).wait()


def _lstm_tc(e_flat, wih_s, whh_s, bias4):
    return pl.pallas_call(
        _lstm_body,
        in_specs=[
            pl.BlockSpec(memory_space=pl.ANY),
            pl.BlockSpec(memory_space=pltpu.MemorySpace.VMEM),
            pl.BlockSpec(memory_space=pltpu.MemorySpace.VMEM),
            pl.BlockSpec(memory_space=pltpu.MemorySpace.VMEM),
        ],
        out_specs=pl.BlockSpec(memory_space=pl.ANY),
        out_shape=jax.ShapeDtypeStruct((B, L, H), jnp.float32),
        scratch_shapes=[
            pltpu.VMEM((3, B, E), jnp.float32),
            pltpu.VMEM((2, B, H), jnp.float32),
            pltpu.SemaphoreType.DMA((3,)),
            pltpu.SemaphoreType.DMA((2,)),
        ],
    )(e_flat, wih_s, whh_s, bias4)


def kernel(x, emb, W_ih, W_hh, b_ih, b_hh):
    e = jnp.take(emb, x.T.reshape(-1), axis=0)
    wih_s = W_ih.reshape(4, H, E).transpose(0, 2, 1).reshape(4 * E, H)
    whh_s = W_hh.reshape(4, H, H).transpose(0, 2, 1).reshape(4 * H, H)
    bias4 = (b_ih + b_hh).reshape(4, H)
    return _lstm_tc(e, wih_s, whh_s, bias4)
